# baseline (speedup 1.0000x reference)
import functools

import jax
import jax.numpy as jnp
from jax import lax
from jax.experimental import pallas as pl
from jax.experimental.pallas import tpu as pltpu
from jax.experimental.pallas import tpu_sc as plsc

_N = 100000
_B = 16384
_NQ2 = _N // 2
_NT8 = _N * 3 // 8

_info = plsc.get_sparse_core_info()
_NC = _info.num_cores
_NS = _info.num_subcores
_NW = _NC * _NS
_BPW = _B // _NW

_mesh = plsc.VectorSubcoreMesh(core_axis_name="c", subcore_axis_name="s")


@functools.partial(
    pl.kernel,
    mesh=_mesh,
    compiler_params=pltpu.CompilerParams(
        use_tc_tiling_on_sc=False, needs_layout_passes=False),
    out_type=jax.ShapeDtypeStruct((_B, 128), jnp.float32),
    scratch_types=[
        pltpu.VMEM((_BPW,), jnp.int32),
        pltpu.VMEM((_BPW,), jnp.int32),
        pltpu.VMEM((_BPW,), jnp.int32),
        pltpu.VMEM((_BPW,), jnp.int32),
        pltpu.VMEM((_BPW, 8), jnp.float32),
        pltpu.VMEM((_BPW, 8), jnp.float32),
        pltpu.VMEM((_BPW, 8), jnp.float32),
        pltpu.VMEM((_BPW, 128), jnp.float32),
        pltpu.SemaphoreType.DMA,
        pltpu.SemaphoreType.DMA,
        pltpu.SemaphoreType.DMA,
    ],
)
def _gather_poses(idx_hbm, big_hbm, out_hbm,
                  idx_v, qi_v, ai_v, bi_v, qp_v, ta_v, tb_v, orow_v,
                  semq, sema, semb):
    wid = lax.axis_index("s") * _NC + lax.axis_index("c")
    base = wid * _BPW
    pltpu.sync_copy(idx_hbm.at[pl.ds(base, _BPW)], idx_v)
    lane = lax.iota(jnp.int32, 16)

    def ibody(c, _):
        iv = idx_v[pl.ds(16 * c, 16)]
        qi_v[pl.ds(16 * c, 16)] = iv >> 1
        a = (3 * iv) >> 3
        ai_v[pl.ds(16 * c, 16)] = a + _NQ2
        bi_v[pl.ds(16 * c, 16)] = jnp.minimum(a + 1, _NT8 - 1) + _NQ2
        return 0

    lax.fori_loop(0, _BPW // 16, ibody, 0)
    cq = pltpu.async_copy(big_hbm.at[qi_v], qp_v, semq)
    ca = pltpu.async_copy(big_hbm.at[ai_v], ta_v, sema)
    cb = pltpu.async_copy(big_hbm.at[bi_v], tb_v, semb)
    cq.wait(); ca.wait(); cb.wait()

    def qbody(c, _):
        m = 16 * c + lane
        k = m >> 2
        j = m & 3
        iv = plsc.load_gather(idx_v, [k])
        vals = plsc.load_gather(qp_v, [k, 4 * (iv & 1) + j])
        plsc.store_scatter(orow_v, [k, j], vals)
        return 0

    def tbody(c, _):
        m = 16 * c + lane
        k = (m * 21846) >> 16
        j = m - 3 * k
        iv = plsc.load_gather(idx_v, [k])
        off = (3 * iv & 7) + j
        va = plsc.load_gather(ta_v, [k, off & 7])
        vb = plsc.load_gather(tb_v, [k, (off - 8) & 7])
        plsc.store_scatter(orow_v, [k, 4 + j], jnp.where(off < 8, va, vb))
        return 0

    lax.fori_loop(0, _BPW * 4 // 16, qbody, 0)
    lax.fori_loop(0, _BPW * 3 // 16, tbody, 0)
    pltpu.sync_copy(orow_v, out_hbm.at[pl.ds(base, _BPW)])


def kernel(camera_pose_indices, q_camera_pointcloud_table,
           t_camera_pointcloud_table):
    idx = camera_pose_indices.astype(jnp.int32)
    big = jnp.concatenate(
        [q_camera_pointcloud_table.reshape(_NQ2, 8),
         t_camera_pointcloud_table.reshape(_NT8, 8)], axis=0)
    out = _gather_poses(idx, big)
    return out[:, :4], out[:, 4:7]


# speedup vs baseline: 2.2877x; 2.2877x over previous
import functools

import jax
import jax.numpy as jnp
from jax import lax
from jax.experimental import pallas as pl
from jax.experimental.pallas import tpu as pltpu
from jax.experimental.pallas import tpu_sc as plsc

_N = 100000
_B = 16384

_info = plsc.get_sparse_core_info()
_NC = _info.num_cores
_NS = _info.num_subcores
_NW = _NC * _NS
_BPW = _B // _NW

_mesh = plsc.VectorSubcoreMesh(core_axis_name="c", subcore_axis_name="s")


@functools.partial(
    pl.kernel,
    mesh=_mesh,
    compiler_params=pltpu.CompilerParams(
        use_tc_tiling_on_sc=False, needs_layout_passes=False),
    out_type=jax.ShapeDtypeStruct((_B, 128), jnp.float32),
    scratch_types=[
        pltpu.VMEM((_BPW,), jnp.int32),
        pltpu.VMEM((_BPW, 8), jnp.float32),
        pltpu.VMEM((_BPW, 128), jnp.float32),
        pltpu.SemaphoreType.DMA,
    ],
)
def _gather_poses(idx_hbm, tab_hbm, out_hbm,
                  idx_v, row_v, orow_v, sem):
    wid = lax.axis_index("s") * _NC + lax.axis_index("c")
    base = wid * _BPW
    pltpu.sync_copy(idx_hbm.at[pl.ds(base, _BPW)], idx_v)
    pltpu.async_copy(tab_hbm.at[idx_v], row_v, sem).wait()
    lane = lax.iota(jnp.int32, 16)

    def rbody(c, _):
        m = 16 * c + lane
        k = m >> 3
        j = m & 7
        vals = plsc.load_gather(row_v, [k, j])
        plsc.store_scatter(orow_v, [k, j], vals)
        return 0

    lax.fori_loop(0, _BPW * 8 // 16, rbody, 0)
    pltpu.sync_copy(orow_v, out_hbm.at[pl.ds(base, _BPW)])


def kernel(camera_pose_indices, q_camera_pointcloud_table,
           t_camera_pointcloud_table):
    idx = camera_pose_indices.astype(jnp.int32)
    fused = jnp.concatenate(
        [q_camera_pointcloud_table,
         t_camera_pointcloud_table,
         jnp.zeros((_N, 1), jnp.float32)], axis=1)
    out = _gather_poses(idx, fused)
    return out[:, :4], out[:, 4:7]


# speedup vs baseline: 5.4456x; 2.3804x over previous
import functools

import jax
import jax.numpy as jnp
from jax import lax
from jax.experimental import pallas as pl
from jax.experimental.pallas import tpu as pltpu
from jax.experimental.pallas import tpu_sc as plsc

_N = 100000
_B = 16384
_RPP = _N // 32

_info = plsc.get_sparse_core_info()
_NC = _info.num_cores
_NS = _info.num_subcores
_NW = _NC * _NS
_BPW = _B // _NW

_mesh = plsc.VectorSubcoreMesh(core_axis_name="c", subcore_axis_name="s")


@functools.partial(
    pl.kernel,
    mesh=_mesh,
    compiler_params=pltpu.CompilerParams(
        use_tc_tiling_on_sc=False, needs_layout_passes=False),
    out_type=(
        jax.ShapeDtypeStruct((4, _B), jnp.float32),
        jax.ShapeDtypeStruct((3, _B), jnp.float32),
    ),
    scratch_types=[
        pltpu.VMEM((_BPW,), jnp.int32),
        pltpu.VMEM((_BPW,), jnp.int32),
        pltpu.VMEM((_BPW, 32), jnp.float32),
        pltpu.VMEM((4 * _BPW,), jnp.float32),
        pltpu.VMEM((3 * _BPW,), jnp.float32),
        pltpu.SemaphoreType.DMA,
    ],
)
def _gather_poses(idx_hbm, q32_hbm, t32_hbm, qT_out, tT_out,
                  idx_v, gi_v, g_v, qT_v, tT_v, sem):
    wid = lax.axis_index("s") * _NC + lax.axis_index("c")
    base = wid * _BPW
    pltpu.sync_copy(idx_hbm.at[pl.ds(base, _BPW)], idx_v)
    lane = lax.iota(jnp.int32, 16)

    def plane(w, tab, dst_v):
        def ib(c, _):
            gi_v[pl.ds(16 * c, 16)] = (
                w * _RPP + (idx_v[pl.ds(16 * c, 16)] >> 5))
            return 0

        lax.fori_loop(0, _BPW // 16, ib, 0)
        pltpu.async_copy(tab.at[gi_v], g_v, sem).wait()

        def rb(c, _):
            kv = 16 * c + lane
            iv = idx_v[pl.ds(16 * c, 16)]
            vals = plsc.load_gather(g_v, [kv, iv & 31])
            dst_v[pl.ds(w * _BPW + 16 * c, 16)] = vals
            return 0

        lax.fori_loop(0, _BPW // 16, rb, 0)

    for w in range(4):
        plane(w, q32_hbm, qT_v)
    for w in range(3):
        plane(w, t32_hbm, tT_v)
    for w in range(4):
        pltpu.sync_copy(qT_v.at[pl.ds(w * _BPW, _BPW)],
                        qT_out.at[w, pl.ds(base, _BPW)])
    for w in range(3):
        pltpu.sync_copy(tT_v.at[pl.ds(w * _BPW, _BPW)],
                        tT_out.at[w, pl.ds(base, _BPW)])


def kernel(camera_pose_indices, q_camera_pointcloud_table,
           t_camera_pointcloud_table):
    idx = camera_pose_indices.astype(jnp.int32)
    q32 = q_camera_pointcloud_table.T.reshape(_N // 8, 32)
    t32 = t_camera_pointcloud_table.T.reshape(_N * 3 // 32, 32)
    qT, tT = _gather_poses(idx, q32, t32)
    return qT.T, tT.T


# speedup vs baseline: 6.3459x; 1.1653x over previous
import functools

import jax
import jax.numpy as jnp
from jax import lax
from jax.experimental import pallas as pl
from jax.experimental.pallas import tpu as pltpu
from jax.experimental.pallas import tpu_sc as plsc

_N = 100000
_B = 16384
_RPP = _N // 32

_info = plsc.get_sparse_core_info()
_NC = _info.num_cores
_NS = _info.num_subcores
_NW = _NC * _NS
_BPW = _B // _NW

_mesh = plsc.VectorSubcoreMesh(core_axis_name="c", subcore_axis_name="s")


@functools.partial(
    pl.kernel,
    mesh=_mesh,
    compiler_params=pltpu.CompilerParams(
        use_tc_tiling_on_sc=False, needs_layout_passes=False),
    out_type=(
        jax.ShapeDtypeStruct((4, _B), jnp.float32),
        jax.ShapeDtypeStruct((3, _B), jnp.float32),
    ),
    scratch_types=[
        pltpu.VMEM((_BPW,), jnp.int32),
        pltpu.VMEM((4 * _BPW,), jnp.int32),
        pltpu.VMEM((3 * _BPW,), jnp.int32),
        pltpu.VMEM((4 * _BPW, 32), jnp.float32),
        pltpu.VMEM((3 * _BPW, 32), jnp.float32),
        pltpu.VMEM((4 * _BPW,), jnp.float32),
        pltpu.VMEM((3 * _BPW,), jnp.float32),
        pltpu.SemaphoreType.DMA,
        pltpu.SemaphoreType.DMA,
    ],
)
def _gather_poses(idx_hbm, q32_hbm, t32_hbm, qT_out, tT_out,
                  idx_v, qgi_v, tgi_v, gq_v, gt_v, qT_v, tT_v,
                  semq, semt):
    wid = lax.axis_index("s") * _NC + lax.axis_index("c")
    base = wid * _BPW
    pltpu.sync_copy(idx_hbm.at[pl.ds(base, _BPW)], idx_v)
    lane = lax.iota(jnp.int32, 16)

    def ib(c, _):
        rows = idx_v[pl.ds(16 * c, 16)] >> 5
        for w in range(4):
            qgi_v[pl.ds(w * _BPW + 16 * c, 16)] = rows + w * _RPP
        for w in range(3):
            tgi_v[pl.ds(w * _BPW + 16 * c, 16)] = rows + w * _RPP
        return 0

    lax.fori_loop(0, _BPW // 16, ib, 0)
    cq = pltpu.async_copy(q32_hbm.at[qgi_v], gq_v, semq)
    ct = pltpu.async_copy(t32_hbm.at[tgi_v], gt_v, semt)
    cq.wait()
    ct.wait()

    def make_rb(g_v, dst_v, w):
        def rb(c, _):
            kv = w * _BPW + 16 * c + lane
            iv = idx_v[pl.ds(16 * c, 16)]
            vals = plsc.load_gather(g_v, [kv, iv & 31])
            dst_v[pl.ds(w * _BPW + 16 * c, 16)] = vals
            return 0
        return rb

    for w in range(4):
        lax.fori_loop(0, _BPW // 16, make_rb(gq_v, qT_v, w), 0)
    for w in range(3):
        lax.fori_loop(0, _BPW // 16, make_rb(gt_v, tT_v, w), 0)
    for w in range(4):
        pltpu.sync_copy(qT_v.at[pl.ds(w * _BPW, _BPW)],
                        qT_out.at[w, pl.ds(base, _BPW)])
    for w in range(3):
        pltpu.sync_copy(tT_v.at[pl.ds(w * _BPW, _BPW)],
                        tT_out.at[w, pl.ds(base, _BPW)])


def kernel(camera_pose_indices, q_camera_pointcloud_table,
           t_camera_pointcloud_table):
    idx = camera_pose_indices.astype(jnp.int32)
    q32 = q_camera_pointcloud_table.T.reshape(_N // 8, 32)
    t32 = t_camera_pointcloud_table.T.reshape(_N * 3 // 32, 32)
    qT, tT = _gather_poses(idx, q32, t32)
    return qT.T, tT.T


# speedup vs baseline: 6.7807x; 1.0685x over previous
import functools

import jax
import jax.numpy as jnp
from jax import lax
from jax.experimental import pallas as pl
from jax.experimental.pallas import tpu as pltpu
from jax.experimental.pallas import tpu_sc as plsc

_N = 100000
_B = 16384
_RPP = _N // 16

_info = plsc.get_sparse_core_info()
_NC = _info.num_cores
_NS = _info.num_subcores
_NW = _NC * _NS
_BPW = _B // _NW

_mesh = plsc.VectorSubcoreMesh(core_axis_name="c", subcore_axis_name="s")


@functools.partial(
    pl.kernel,
    mesh=_mesh,
    compiler_params=pltpu.CompilerParams(
        use_tc_tiling_on_sc=False, needs_layout_passes=False),
    out_type=(
        jax.ShapeDtypeStruct((4, _B), jnp.float32),
        jax.ShapeDtypeStruct((3, _B), jnp.float32),
    ),
    scratch_types=[
        pltpu.VMEM((_BPW,), jnp.int32),
        pltpu.VMEM((4 * _BPW,), jnp.int32),
        pltpu.VMEM((3 * _BPW,), jnp.int32),
        pltpu.VMEM((4 * _BPW, 16), jnp.float32),
        pltpu.VMEM((3 * _BPW, 16), jnp.float32),
        pltpu.VMEM((4 * _BPW,), jnp.float32),
        pltpu.VMEM((3 * _BPW,), jnp.float32),
        pltpu.SemaphoreType.DMA,
        pltpu.SemaphoreType.DMA,
        pltpu.SemaphoreType.DMA,
    ],
)
def _gather_poses(idx_hbm, q16_hbm, t16_hbm, qT_out, tT_out,
                  idx_v, qgi_v, tgi_v, gq_v, gt_v, qT_v, tT_v,
                  sem1, sem2, sem3):
    wid = lax.axis_index("s") * _NC + lax.axis_index("c")
    base = wid * _BPW
    pltpu.sync_copy(idx_hbm.at[pl.ds(base, _BPW)], idx_v)
    lane = lax.iota(jnp.int32, 16)

    def ib(c, _):
        rows = idx_v[pl.ds(16 * c, 16)] >> 4
        for w in range(4):
            qgi_v[pl.ds(w * _BPW + 16 * c, 16)] = rows + w * _RPP
        for w in range(3):
            tgi_v[pl.ds(w * _BPW + 16 * c, 16)] = rows + w * _RPP
        return 0

    lax.fori_loop(0, _BPW // 16, ib, 0)
    c1 = pltpu.async_copy(q16_hbm.at[qgi_v.at[pl.ds(0, 2 * _BPW)]],
                          gq_v.at[pl.ds(0, 2 * _BPW)], sem1)
    c2 = pltpu.async_copy(q16_hbm.at[qgi_v.at[pl.ds(2 * _BPW, 2 * _BPW)]],
                          gq_v.at[pl.ds(2 * _BPW, 2 * _BPW)], sem2)
    c3 = pltpu.async_copy(t16_hbm.at[tgi_v], gt_v, sem3)

    def make_rb(g_v, dst_v, w):
        def rb(c, _):
            kv = w * _BPW + 16 * c + lane
            iv = idx_v[pl.ds(16 * c, 16)]
            vals = plsc.load_gather(g_v, [kv, iv & 15])
            dst_v[pl.ds(w * _BPW + 16 * c, 16)] = vals
            return 0
        return rb

    c1.wait()
    for w in range(2):
        lax.fori_loop(0, _BPW // 16, make_rb(gq_v, qT_v, w), 0)
    c2.wait()
    for w in range(2, 4):
        lax.fori_loop(0, _BPW // 16, make_rb(gq_v, qT_v, w), 0)
    c3.wait()
    for w in range(3):
        lax.fori_loop(0, _BPW // 16, make_rb(gt_v, tT_v, w), 0)
    for w in range(4):
        pltpu.sync_copy(qT_v.at[pl.ds(w * _BPW, _BPW)],
                        qT_out.at[w, pl.ds(base, _BPW)])
    for w in range(3):
        pltpu.sync_copy(tT_v.at[pl.ds(w * _BPW, _BPW)],
                        tT_out.at[w, pl.ds(base, _BPW)])


def kernel(camera_pose_indices, q_camera_pointcloud_table,
           t_camera_pointcloud_table):
    idx = camera_pose_indices.astype(jnp.int32)
    q16 = q_camera_pointcloud_table.T.reshape(_N // 4, 16)
    t16 = t_camera_pointcloud_table.T.reshape(_N * 3 // 16, 16)
    qT, tT = _gather_poses(idx, q16, t16)
    return qT.T, tT.T


# speedup vs baseline: 7.0968x; 1.0466x over previous
import functools

import jax
import jax.numpy as jnp
from jax import lax
from jax.experimental import pallas as pl
from jax.experimental.pallas import tpu as pltpu
from jax.experimental.pallas import tpu_sc as plsc

_N = 100000
_B = 16384
_RPP = _N // 16

_info = plsc.get_sparse_core_info()
_NC = _info.num_cores
_NS = _info.num_subcores
_NW = _NC * _NS
_BPW = _B // _NW

_mesh = plsc.VectorSubcoreMesh(core_axis_name="c", subcore_axis_name="s")


@functools.partial(
    pl.kernel,
    mesh=_mesh,
    compiler_params=pltpu.CompilerParams(
        use_tc_tiling_on_sc=False, needs_layout_passes=False),
    out_type=(
        jax.ShapeDtypeStruct((_B // 128, 4, 128), jnp.float32),
        jax.ShapeDtypeStruct((_B // 128, 3, 128), jnp.float32),
    ),
    scratch_types=[
        pltpu.VMEM((_BPW,), jnp.int32),
        pltpu.VMEM((4 * _BPW,), jnp.int32),
        pltpu.VMEM((3 * _BPW,), jnp.int32),
        pltpu.VMEM((4 * _BPW, 16), jnp.float32),
        pltpu.VMEM((3 * _BPW, 16), jnp.float32),
        pltpu.VMEM((4, _BPW), jnp.float32),
        pltpu.VMEM((3, _BPW), jnp.float32),
        pltpu.SemaphoreType.DMA,
        pltpu.SemaphoreType.DMA,
        pltpu.SemaphoreType.DMA,
    ],
)
def _gather_poses(idx_hbm, q16_hbm, t16_hbm, qT_out, tT_out,
                  idx_v, qgi_v, tgi_v, gq_v, gt_v, qT_v, tT_v,
                  sem1, sem2, sem3):
    wid = lax.axis_index("s") * _NC + lax.axis_index("c")
    base = wid * _BPW
    pltpu.sync_copy(idx_hbm.at[pl.ds(base, _BPW)], idx_v)
    lane = lax.iota(jnp.int32, 16)

    def ib(c, _):
        rows = idx_v[pl.ds(16 * c, 16)] >> 4
        for w in range(4):
            qgi_v[pl.ds(w * _BPW + 16 * c, 16)] = rows + w * _RPP
        for w in range(3):
            tgi_v[pl.ds(w * _BPW + 16 * c, 16)] = rows + w * _RPP
        return 0

    lax.fori_loop(0, _BPW // 16, ib, 0)
    c1 = pltpu.async_copy(q16_hbm.at[qgi_v.at[pl.ds(0, 2 * _BPW)]],
                          gq_v.at[pl.ds(0, 2 * _BPW)], sem1)
    c2 = pltpu.async_copy(q16_hbm.at[qgi_v.at[pl.ds(2 * _BPW, 2 * _BPW)]],
                          gq_v.at[pl.ds(2 * _BPW, 2 * _BPW)], sem2)
    c3 = pltpu.async_copy(t16_hbm.at[tgi_v], gt_v, sem3)

    def make_rb(g_v, dst_v, w):
        def rb(c, _):
            kv = w * _BPW + 16 * c + lane
            iv = idx_v[pl.ds(16 * c, 16)]
            vals = plsc.load_gather(g_v, [kv, iv & 15])
            dst_v[w, pl.ds(16 * c, 16)] = vals
            return 0
        return rb

    c1.wait()
    for w in range(2):
        lax.fori_loop(0, _BPW // 16, make_rb(gq_v, qT_v, w), 0)
    c2.wait()
    for w in range(2, 4):
        lax.fori_loop(0, _BPW // 16, make_rb(gq_v, qT_v, w), 0)
    c3.wait()
    for w in range(3):
        lax.fori_loop(0, _BPW // 16, make_rb(gt_v, tT_v, w), 0)
    cbase = base // 128
    for ch in range(_BPW // 128):
        pltpu.sync_copy(qT_v.at[:, pl.ds(128 * ch, 128)],
                        qT_out.at[cbase + ch])
        pltpu.sync_copy(tT_v.at[:, pl.ds(128 * ch, 128)],
                        tT_out.at[cbase + ch])


def kernel(camera_pose_indices, q_camera_pointcloud_table,
           t_camera_pointcloud_table):
    idx = camera_pose_indices.astype(jnp.int32)
    q16 = q_camera_pointcloud_table.T.reshape(_N // 4, 16)
    t16 = t_camera_pointcloud_table.T.reshape(_N * 3 // 16, 16)
    qc, tc = _gather_poses(idx, q16, t16)
    return (qc.transpose(0, 2, 1).reshape(_B, 4),
            tc.transpose(0, 2, 1).reshape(_B, 3))


# speedup vs baseline: 7.5406x; 1.0625x over previous
import functools

import jax
import jax.numpy as jnp
from jax import lax
from jax.experimental import pallas as pl
from jax.experimental.pallas import tpu as pltpu
from jax.experimental.pallas import tpu_sc as plsc

_N = 100000
_B = 16384
_RPP = _N // 16

_info = plsc.get_sparse_core_info()
_NC = _info.num_cores
_NS = _info.num_subcores
_NW = _NC * _NS
_BPW = _B // _NW

_mesh = plsc.VectorSubcoreMesh(core_axis_name="c", subcore_axis_name="s")


@functools.partial(
    pl.kernel,
    mesh=_mesh,
    compiler_params=pltpu.CompilerParams(
        use_tc_tiling_on_sc=False, needs_layout_passes=False),
    out_type=(
        jax.ShapeDtypeStruct((_B // 128, 4, 128), jnp.float32),
        jax.ShapeDtypeStruct((_B // 128, 4, 128), jnp.float32),
    ),
    scratch_types=[
        pltpu.VMEM((_BPW,), jnp.int32),
        pltpu.VMEM((4 * _BPW,), jnp.int32),
        pltpu.VMEM((3 * _BPW,), jnp.int32),
        pltpu.VMEM((4 * _BPW, 16), jnp.float32),
        pltpu.VMEM((3 * _BPW, 16), jnp.float32),
        pltpu.VMEM((4, _BPW), jnp.float32),
        pltpu.VMEM((4, _BPW), jnp.float32),
        pltpu.SemaphoreType.DMA,
        pltpu.SemaphoreType.DMA,
        pltpu.SemaphoreType.DMA,
    ],
)
def _gather_poses(idx_hbm, q16_hbm, t16_hbm, qT_out, tT_out,
                  idx_v, qgi_v, tgi_v, gq_v, gt_v, qT_v, tT_v,
                  sem1, sem2, sem3):
    wid = lax.axis_index("s") * _NC + lax.axis_index("c")
    base = wid * _BPW
    pltpu.sync_copy(idx_hbm.at[pl.ds(base, _BPW)], idx_v)
    lane = lax.iota(jnp.int32, 16)

    def ib1(c, _):
        rows = idx_v[pl.ds(16 * c, 16)] >> 4
        for w in range(2):
            qgi_v[pl.ds(w * _BPW + 16 * c, 16)] = rows + w * _RPP
        return 0

    def ib2(c, _):
        rows = idx_v[pl.ds(16 * c, 16)] >> 4
        for w in range(2, 4):
            qgi_v[pl.ds(w * _BPW + 16 * c, 16)] = rows + w * _RPP
        for w in range(3):
            tgi_v[pl.ds(w * _BPW + 16 * c, 16)] = rows + w * _RPP
        return 0

    lax.fori_loop(0, _BPW // 16, ib1, 0)
    c1 = pltpu.async_copy(q16_hbm.at[qgi_v.at[pl.ds(0, 2 * _BPW)]],
                          gq_v.at[pl.ds(0, 2 * _BPW)], sem1)
    lax.fori_loop(0, _BPW // 16, ib2, 0)
    c2 = pltpu.async_copy(q16_hbm.at[qgi_v.at[pl.ds(2 * _BPW, 2 * _BPW)]],
                          gq_v.at[pl.ds(2 * _BPW, 2 * _BPW)], sem2)
    c3 = pltpu.async_copy(t16_hbm.at[tgi_v], gt_v, sem3)

    def make_rb(g_v, dst_v, w):
        def rb(c, _):
            kv = w * _BPW + 16 * c + lane
            iv = idx_v[pl.ds(16 * c, 16)]
            vals = plsc.load_gather(g_v, [kv, iv & 15])
            dst_v[w, pl.ds(16 * c, 16)] = vals
            return 0
        return rb

    c1.wait()
    for w in range(2):
        lax.fori_loop(0, _BPW // 16, make_rb(gq_v, qT_v, w), 0)
    c2.wait()
    for w in range(2, 4):
        lax.fori_loop(0, _BPW // 16, make_rb(gq_v, qT_v, w), 0)
    c3.wait()
    for w in range(3):
        lax.fori_loop(0, _BPW // 16, make_rb(gt_v, tT_v, w), 0)
    cbase = base // 128
    for ch in range(_BPW // 128):
        pltpu.sync_copy(qT_v.at[:, pl.ds(128 * ch, 128)],
                        qT_out.at[cbase + ch])
        pltpu.sync_copy(tT_v.at[:, pl.ds(128 * ch, 128)],
                        tT_out.at[cbase + ch])


def kernel(camera_pose_indices, q_camera_pointcloud_table,
           t_camera_pointcloud_table):
    idx = camera_pose_indices.astype(jnp.int32)
    q16 = q_camera_pointcloud_table.T.reshape(_N // 4, 16)
    t16 = t_camera_pointcloud_table.T.reshape(_N * 3 // 16, 16)
    qc, tc = _gather_poses(idx, q16, t16)
    return (qc.transpose(0, 2, 1).reshape(_B, 4),
            tc.transpose(0, 2, 1).reshape(_B, 4)[:, :3])


# speedup vs baseline: 7.6302x; 1.0119x over previous
import functools

import jax
import jax.numpy as jnp
from jax import lax
from jax.experimental import pallas as pl
from jax.experimental.pallas import tpu as pltpu
from jax.experimental.pallas import tpu_sc as plsc

_N = 100000
_B = 16384
_RPP = _N // 16

_info = plsc.get_sparse_core_info()
_NC = _info.num_cores
_NS = _info.num_subcores
_NW = _NC * _NS
_BPW = _B // _NW

_mesh = plsc.VectorSubcoreMesh(core_axis_name="c", subcore_axis_name="s")


@functools.partial(
    pl.kernel,
    mesh=_mesh,
    compiler_params=pltpu.CompilerParams(
        use_tc_tiling_on_sc=False, needs_layout_passes=False),
    out_type=(
        jax.ShapeDtypeStruct((_B // 128, 4, 128), jnp.float32),
        jax.ShapeDtypeStruct((_B // 128, 4, 128), jnp.float32),
    ),
    scratch_types=[
        pltpu.VMEM((_BPW,), jnp.int32),
        pltpu.VMEM((4 * _BPW,), jnp.int32),
        pltpu.VMEM((3 * _BPW,), jnp.int32),
        pltpu.VMEM((4 * _BPW, 16), jnp.float32),
        pltpu.VMEM((3 * _BPW, 16), jnp.float32),
        pltpu.VMEM((4, _BPW), jnp.float32),
        pltpu.VMEM((4, _BPW), jnp.float32),
        pltpu.SemaphoreType.DMA,
        pltpu.SemaphoreType.DMA,
        pltpu.SemaphoreType.DMA,
    ],
)
def _gather_poses(idx_hbm, qt16_hbm, qT_out, tT_out,
                  idx_v, qgi_v, tgi_v, gq_v, gt_v, qT_v, tT_v,
                  sem1, sem2, sem3):
    wid = lax.axis_index("s") * _NC + lax.axis_index("c")
    base = wid * _BPW
    pltpu.sync_copy(idx_hbm.at[pl.ds(base, _BPW)], idx_v)
    lane = lax.iota(jnp.int32, 16)

    def ib1(c, _):
        rows = idx_v[pl.ds(16 * c, 16)] >> 4
        for w in range(2):
            qgi_v[pl.ds(w * _BPW + 16 * c, 16)] = rows + w * _RPP
        return 0

    def ib2(c, _):
        rows = idx_v[pl.ds(16 * c, 16)] >> 4
        for w in range(2, 4):
            qgi_v[pl.ds(w * _BPW + 16 * c, 16)] = rows + w * _RPP
        for w in range(3):
            tgi_v[pl.ds(w * _BPW + 16 * c, 16)] = (
                rows + (_N // 4 + w * _RPP))
        return 0

    lax.fori_loop(0, _BPW // 16, ib1, 0)
    c1 = pltpu.async_copy(qt16_hbm.at[qgi_v.at[pl.ds(0, 2 * _BPW)]],
                          gq_v.at[pl.ds(0, 2 * _BPW)], sem1)
    lax.fori_loop(0, _BPW // 16, ib2, 0)
    c2 = pltpu.async_copy(qt16_hbm.at[qgi_v.at[pl.ds(2 * _BPW, 2 * _BPW)]],
                          gq_v.at[pl.ds(2 * _BPW, 2 * _BPW)], sem2)
    c3 = pltpu.async_copy(qt16_hbm.at[tgi_v], gt_v, sem3)

    def make_rb(g_v, dst_v, w):
        def rb(c, _):
            kv = w * _BPW + 16 * c + lane
            iv = idx_v[pl.ds(16 * c, 16)]
            vals = plsc.load_gather(g_v, [kv, iv & 15])
            dst_v[w, pl.ds(16 * c, 16)] = vals
            return 0
        return rb

    c1.wait()
    for w in range(2):
        lax.fori_loop(0, _BPW // 16, make_rb(gq_v, qT_v, w), 0)
    c2.wait()
    for w in range(2, 4):
        lax.fori_loop(0, _BPW // 16, make_rb(gq_v, qT_v, w), 0)
    c3.wait()
    for w in range(3):
        lax.fori_loop(0, _BPW // 16, make_rb(gt_v, tT_v, w), 0)
    cbase = base // 128
    for ch in range(_BPW // 128):
        pltpu.sync_copy(qT_v.at[:, pl.ds(128 * ch, 128)],
                        qT_out.at[cbase + ch])
        pltpu.sync_copy(tT_v.at[:, pl.ds(128 * ch, 128)],
                        tT_out.at[cbase + ch])


def kernel(camera_pose_indices, q_camera_pointcloud_table,
           t_camera_pointcloud_table):
    idx = camera_pose_indices.astype(jnp.int32)
    qt16 = jnp.concatenate(
        [q_camera_pointcloud_table.T.reshape(4 * _N),
         t_camera_pointcloud_table.T.reshape(3 * _N)]).reshape(-1, 16)
    qc, tc = _gather_poses(idx, qt16)
    return (qc.transpose(0, 2, 1).reshape(_B, 4),
            tc.transpose(0, 2, 1).reshape(_B, 4)[:, :3])


# speedup vs baseline: 7.8302x; 1.0262x over previous
import functools

import jax
import jax.numpy as jnp
from jax import lax
from jax.experimental import pallas as pl
from jax.experimental.pallas import tpu as pltpu
from jax.experimental.pallas import tpu_sc as plsc

_N = 100000
_B = 16384
_RPP = _N // 16

_info = plsc.get_sparse_core_info()
_NC = _info.num_cores
_NS = _info.num_subcores
_NW = _NC * _NS
_BPW = _B // _NW

_mesh = plsc.VectorSubcoreMesh(core_axis_name="c", subcore_axis_name="s")


@functools.partial(
    pl.kernel,
    mesh=_mesh,
    compiler_params=pltpu.CompilerParams(
        use_tc_tiling_on_sc=False, needs_layout_passes=False),
    out_type=(
        jax.ShapeDtypeStruct((_B // 128, 4, 128), jnp.float32),
        jax.ShapeDtypeStruct((_B // 128, 4, 128), jnp.float32),
    ),
    scratch_types=[
        pltpu.VMEM((_BPW,), jnp.int32),
        pltpu.VMEM((4 * _BPW,), jnp.int32),
        pltpu.VMEM((3 * _BPW,), jnp.int32),
        pltpu.VMEM((4 * _BPW, 16), jnp.float32),
        pltpu.VMEM((3 * _BPW, 16), jnp.float32),
        pltpu.VMEM((4, _BPW), jnp.float32),
        pltpu.VMEM((4, _BPW), jnp.float32),
        pltpu.SemaphoreType.DMA,
        pltpu.SemaphoreType.DMA,
        pltpu.SemaphoreType.DMA,
    ],
)
def _gather_poses(idx_hbm, qt16_hbm, qT_out, tT_out,
                  idx_v, qgi_v, tgi_v, gq_v, gt_v, qT_v, tT_v,
                  sem1, sem2, sem3):
    wid = lax.axis_index("s") * _NC + lax.axis_index("c")
    base = wid * _BPW
    pltpu.sync_copy(idx_hbm.at[pl.ds(base, _BPW)], idx_v)
    lane = lax.iota(jnp.int32, 16)

    def ib1(c, _):
        rows = idx_v[pl.ds(16 * c, 16)] >> 4
        for w in range(2):
            qgi_v[pl.ds(w * _BPW + 16 * c, 16)] = rows + w * _RPP
        return 0

    def ib2(c, _):
        rows = idx_v[pl.ds(16 * c, 16)] >> 4
        for w in range(2, 4):
            qgi_v[pl.ds(w * _BPW + 16 * c, 16)] = rows + w * _RPP
        for w in range(3):
            tgi_v[pl.ds(w * _BPW + 16 * c, 16)] = (
                rows + (_N // 4 + w * _RPP))
        return 0

    lax.fori_loop(0, _BPW // 16, ib1, 0)
    c1 = pltpu.async_copy(qt16_hbm.at[qgi_v.at[pl.ds(0, 2 * _BPW)]],
                          gq_v.at[pl.ds(0, 2 * _BPW)], sem1)
    lax.fori_loop(0, _BPW // 16, ib2, 0)
    c2 = pltpu.async_copy(qt16_hbm.at[qgi_v.at[pl.ds(2 * _BPW, 2 * _BPW)]],
                          gq_v.at[pl.ds(2 * _BPW, 2 * _BPW)], sem2)
    c3 = pltpu.async_copy(qt16_hbm.at[tgi_v], gt_v, sem3)

    def make_rb(g_v, dst_v, ws):
        def rb(c, _):
            iv = idx_v[pl.ds(16 * c, 16)]
            col = iv & 15
            for w in ws:
                kv = w * _BPW + 16 * c + lane
                dst_v[w, pl.ds(16 * c, 16)] = (
                    plsc.load_gather(g_v, [kv, col]))
            return 0
        return rb

    c1.wait()
    lax.fori_loop(0, _BPW // 16, make_rb(gq_v, qT_v, (0, 1)), 0)
    c2.wait()
    lax.fori_loop(0, _BPW // 16, make_rb(gq_v, qT_v, (2, 3)), 0)
    c3.wait()
    lax.fori_loop(0, _BPW // 16, make_rb(gt_v, tT_v, (0, 1, 2)), 0)
    cbase = base // 128
    for ch in range(_BPW // 128):
        pltpu.sync_copy(qT_v.at[:, pl.ds(128 * ch, 128)],
                        qT_out.at[cbase + ch])
        pltpu.sync_copy(tT_v.at[:, pl.ds(128 * ch, 128)],
                        tT_out.at[cbase + ch])


def kernel(camera_pose_indices, q_camera_pointcloud_table,
           t_camera_pointcloud_table):
    idx = camera_pose_indices.astype(jnp.int32)
    qt16 = jnp.concatenate(
        [q_camera_pointcloud_table.T.reshape(4 * _N),
         t_camera_pointcloud_table.T.reshape(3 * _N)]).reshape(-1, 16)
    qc, tc = _gather_poses(idx, qt16)
    return (qc.transpose(0, 2, 1).reshape(_B, 4),
            tc.transpose(0, 2, 1).reshape(_B, 4)[:, :3])
